# baseline (device time: 87557 ns/iter reference)
import jax
import jax.numpy as jnp
from jax import lax
from jax.experimental import pallas as pl
from jax.experimental.pallas import tpu as pltpu

N_DEV = 4
N_HOP = N_DEV - 1
N_SUB = 8


def kernel(x, w_mat, scale_x, scale_w):
    m_per, k = x.shape
    _, n_per = w_mat.shape
    half = m_per // 2
    sub = half // N_SUB

    def body(x_hbm, w_ref, sx_ref, sw_ref, out_ref,
             xstage_ref, w8_ref, cw_ref, ccw_ref,
             cw_send, cw_recv, ccw_send, ccw_recv, x_sems):
        my = lax.axis_index("i")
        left = lax.rem(my + N_DEV - 1, N_DEV)
        right = lax.rem(my + 1, N_DEV)

        def rdma(ref, h, s, sems, dev):
            src = N_HOP - 1 if h == 0 else h - 1
            return pltpu.make_async_remote_copy(
                src_ref=ref.at[src, pl.ds(s * sub, sub)],
                dst_ref=ref.at[h, pl.ds(s * sub, sub)],
                send_sem=sems[0].at[h, s],
                recv_sem=sems[1].at[h, s],
                device_id=(dev,), device_id_type=pl.DeviceIdType.MESH,
            )

        cw = lambda h, s: rdma(cw_ref, h, s, (cw_send, cw_recv), right)
        ccw = lambda h, s: rdma(ccw_ref, h, s, (ccw_send, ccw_recv), left)

        n_x = 2 * N_SUB

        def xdma(c, stage):
            row = (c % 2) * half + (c // 2) * sub
            return pltpu.make_async_copy(
                x_hbm.at[pl.ds(row, sub)], xstage_ref.at[stage], x_sems.at[c])

        for c in (0, 1):
            xdma(c, c).start()

        barrier_sem = pltpu.get_barrier_semaphore()
        for nbr in (left, right):
            pl.semaphore_signal(
                barrier_sem, inc=1,
                device_id=(nbr,), device_id_type=pl.DeviceIdType.MESH,
            )
        pl.semaphore_wait(barrier_sem, 2)

        wchunk = k // n_x

        for c in range(n_x):
            s = c // 2
            ref, descr = (cw_ref, cw(0, s)) if c % 2 == 0 else (ccw_ref, ccw(0, s))
            xdma(c, c % 2).wait()
            ref[N_HOP - 1, pl.ds(s * sub, sub)] = (
                xstage_ref[c % 2].astype(jnp.float8_e4m3fn))
            descr.start()
            if c + 2 < n_x:
                xdma(c + 2, c % 2).start()
            w_sl = pl.ds(c * wchunk, wchunk)
            w8_ref[w_sl] = w_ref[w_sl].astype(jnp.float8_e4m3fn)

        s_deq = sx_ref[0] * sw_ref[0]

        def gemm(src, origin, top, s=None):
            row = origin * m_per + top * half
            if s is not None:
                row = row + s * sub
            acc = jax.lax.dot_general(
                src, w8_ref[...],
                dimension_numbers=(((1,), (0,)), ((), ())),
                preferred_element_type=jnp.float32,
            )
            out_ref[pl.ds(row, src.shape[0]), :] = (
                jnp.maximum(acc * s_deq, 0.0))

        def compute(h):
            slot = N_HOP - 1 if h == 0 else h - 1
            gemm(cw_ref[slot], lax.rem(my - h + N_DEV, N_DEV), 0)
            gemm(ccw_ref[slot], lax.rem(my + h, N_DEV), 1)

        compute(0)

        for h in range(1, N_HOP + 1):
            last = h == N_HOP
            for s in range(N_SUB):
                cw(h - 1, s).wait_recv()
                ccw(h - 1, s).wait_recv()
                if not last:
                    cw(h, s).start()
                    ccw(h, s).start()
                else:
                    sl = pl.ds(s * sub, sub)
                    gemm(cw_ref[h - 1, sl], lax.rem(my - h + N_DEV, N_DEV), 0, s)
                    gemm(ccw_ref[h - 1, sl], lax.rem(my + h, N_DEV), 1, s)
            if not last:
                compute(h)

        for h in range(N_HOP):
            for s in range(N_SUB):
                cw(h, s).wait_send()
                ccw(h, s).wait_send()

    return pl.pallas_call(
        body,
        out_shape=jax.ShapeDtypeStruct((N_DEV * m_per, n_per), jnp.float32),
        in_specs=[
            pl.BlockSpec(memory_space=pl.ANY),
            pl.BlockSpec(memory_space=pltpu.VMEM),
            pl.BlockSpec(memory_space=pltpu.SMEM),
            pl.BlockSpec(memory_space=pltpu.SMEM),
        ],
        out_specs=pl.BlockSpec(memory_space=pltpu.VMEM),
        scratch_shapes=[
            pltpu.VMEM((2, sub, k), jnp.float32),
            pltpu.VMEM((k, n_per), jnp.float8_e4m3fn),
            pltpu.VMEM((N_HOP, half, k), jnp.float8_e4m3fn),
            pltpu.VMEM((N_HOP, half, k), jnp.float8_e4m3fn),
            pltpu.SemaphoreType.DMA((N_HOP, N_SUB)),
            pltpu.SemaphoreType.DMA((N_HOP, N_SUB)),
            pltpu.SemaphoreType.DMA((N_HOP, N_SUB)),
            pltpu.SemaphoreType.DMA((N_HOP, N_SUB)),
            pltpu.SemaphoreType.DMA((2 * N_SUB,)),
        ],
        compiler_params=pltpu.CompilerParams(
            collective_id=0, vmem_limit_bytes=100 * 1024 * 1024),
    )(x, w_mat, scale_x, scale_w)


# device time: 82046 ns/iter; 1.0672x vs baseline; 1.0672x over previous
import jax
import jax.numpy as jnp
from jax import lax
from jax.experimental import pallas as pl
from jax.experimental.pallas import tpu as pltpu

N_DEV = 4
N_HOP = N_DEV - 1
N_SUB = 8


def kernel(x, w_mat, scale_x, scale_w):
    m_per, k = x.shape
    _, n_per = w_mat.shape
    half = m_per // 2
    sub = half // N_SUB

    w8 = w_mat.astype(jnp.float8_e4m3fn)

    def body(x_hbm, w8_ref, sx_ref, sw_ref, out_ref,
             xstage_ref, cw_ref, ccw_ref,
             cw_send, cw_recv, ccw_send, ccw_recv, x_sems):
        my = lax.axis_index("i")
        left = lax.rem(my + N_DEV - 1, N_DEV)
        right = lax.rem(my + 1, N_DEV)

        def rdma(ref, h, s, sems, dev):
            src = N_HOP - 1 if h == 0 else h - 1
            return pltpu.make_async_remote_copy(
                src_ref=ref.at[src, pl.ds(s * sub, sub)],
                dst_ref=ref.at[h, pl.ds(s * sub, sub)],
                send_sem=sems[0].at[h, s],
                recv_sem=sems[1].at[h, s],
                device_id=(dev,), device_id_type=pl.DeviceIdType.MESH,
            )

        cw = lambda h, s: rdma(cw_ref, h, s, (cw_send, cw_recv), right)
        ccw = lambda h, s: rdma(ccw_ref, h, s, (ccw_send, ccw_recv), left)

        n_x = 2 * N_SUB

        def xdma(c, stage):
            row = (c % 2) * half + (c // 2) * sub
            return pltpu.make_async_copy(
                x_hbm.at[pl.ds(row, sub)], xstage_ref.at[stage], x_sems.at[c])

        for c in (0, 1):
            xdma(c, c).start()

        barrier_sem = pltpu.get_barrier_semaphore()
        for nbr in (left, right):
            pl.semaphore_signal(
                barrier_sem, inc=1,
                device_id=(nbr,), device_id_type=pl.DeviceIdType.MESH,
            )
        pl.semaphore_wait(barrier_sem, 2)

        for c in range(n_x):
            s = c // 2
            ref, descr = (cw_ref, cw(0, s)) if c % 2 == 0 else (ccw_ref, ccw(0, s))
            xdma(c, c % 2).wait()
            ref[N_HOP - 1, pl.ds(s * sub, sub)] = (
                xstage_ref[c % 2].astype(jnp.float8_e4m3fn))
            descr.start()
            if c + 2 < n_x:
                xdma(c + 2, c % 2).start()

        s_deq = sx_ref[0] * sw_ref[0]

        def gemm(src, origin, top, s=None):
            row = origin * m_per + top * half
            if s is not None:
                row = row + s * sub
            acc = jax.lax.dot_general(
                src, w8_ref[...],
                dimension_numbers=(((1,), (0,)), ((), ())),
                preferred_element_type=jnp.float32,
            )
            out_ref[pl.ds(row, src.shape[0]), :] = (
                jnp.maximum(acc * s_deq, 0.0))

        def compute(h):
            slot = N_HOP - 1 if h == 0 else h - 1
            gemm(cw_ref[slot], lax.rem(my - h + N_DEV, N_DEV), 0)
            gemm(ccw_ref[slot], lax.rem(my + h, N_DEV), 1)

        compute(0)

        for h in range(1, N_HOP + 1):
            last = h == N_HOP
            for s in range(N_SUB):
                cw(h - 1, s).wait_recv()
                ccw(h - 1, s).wait_recv()
                if not last:
                    cw(h, s).start()
                    ccw(h, s).start()
                else:
                    sl = pl.ds(s * sub, sub)
                    gemm(cw_ref[h - 1, sl], lax.rem(my - h + N_DEV, N_DEV), 0, s)
                    gemm(ccw_ref[h - 1, sl], lax.rem(my + h, N_DEV), 1, s)
            if not last:
                compute(h)

        for h in range(N_HOP):
            for s in range(N_SUB):
                cw(h, s).wait_send()
                ccw(h, s).wait_send()

    return pl.pallas_call(
        body,
        out_shape=jax.ShapeDtypeStruct((N_DEV * m_per, n_per), jnp.float32),
        in_specs=[
            pl.BlockSpec(memory_space=pl.ANY),
            pl.BlockSpec(memory_space=pltpu.VMEM),
            pl.BlockSpec(memory_space=pltpu.SMEM),
            pl.BlockSpec(memory_space=pltpu.SMEM),
        ],
        out_specs=pl.BlockSpec(memory_space=pltpu.VMEM),
        scratch_shapes=[
            pltpu.VMEM((2, sub, k), jnp.float32),
            pltpu.VMEM((N_HOP, half, k), jnp.float8_e4m3fn),
            pltpu.VMEM((N_HOP, half, k), jnp.float8_e4m3fn),
            pltpu.SemaphoreType.DMA((N_HOP, N_SUB)),
            pltpu.SemaphoreType.DMA((N_HOP, N_SUB)),
            pltpu.SemaphoreType.DMA((N_HOP, N_SUB)),
            pltpu.SemaphoreType.DMA((N_HOP, N_SUB)),
            pltpu.SemaphoreType.DMA((2 * N_SUB,)),
        ],
        compiler_params=pltpu.CompilerParams(
            collective_id=0,
            allow_input_fusion=[False, True, False, False],
        ),
    )(x, w8, scale_x, scale_w)
